# baseline (speedup 1.0000x reference)
import functools

import jax
import jax.numpy as jnp
from jax import lax
from jax.experimental import pallas as pl
from jax.experimental.pallas import tpu as pltpu
from jax.experimental.pallas import tpu_sc as plsc

USED_ROWS = 500000
PAIR_ROWS = USED_ROWS // 2
LANES = 16
CHUNK = 128


def _mesh_info():
  info = plsc.get_sparse_core_info()
  nw = info.num_cores * info.num_subcores
  mesh = plsc.VectorSubcoreMesh(core_axis_name="c", subcore_axis_name="s")
  return info, nw, mesh


@functools.cache
def _build_pack(embed_dim: int):
  info, nw, mesh = _mesh_info()
  d2 = 2 * embed_dim
  q_tiles = PAIR_ROWS // 8 // nw
  extra = PAIR_ROWS // 8 - nw * q_tiles
  c_pairs = 128
  c_rows = 2 * c_pairs
  n_chunks = q_tiles // 16

  @functools.partial(
      pl.kernel,
      mesh=mesh,
      out_type=jax.ShapeDtypeStruct((PAIR_ROWS, d2), jnp.float32),
      scratch_types=[
          pltpu.VMEM((2, c_rows, embed_dim), jnp.float32),
          pltpu.VMEM((2, c_pairs, d2), jnp.float32),
          [pltpu.SemaphoreType.DMA] * 2,
          [pltpu.SemaphoreType.DMA] * 2,
      ],
  )
  def pack_kernel(table_hbm, pairs_hbm, bufa, bufb, in_sems, out_sems):
    wid = lax.axis_index("s") * info.num_cores + lax.axis_index("c")
    base_d = 8 * (wid * q_tiles + jnp.minimum(wid, extra))
    base_w = 2 * base_d

    def start_in(j, b):
      pltpu.async_copy(
          table_hbm.at[pl.ds(base_w + j * c_rows, c_rows)],
          bufa.at[b],
          in_sems[b],
      )

    def wait_in(b):
      pltpu.make_async_copy(
          table_hbm.at[pl.ds(base_w, c_rows)], bufa.at[b], in_sems[b]
      ).wait()

    def start_out(j, b):
      pltpu.async_copy(
          bufb.at[b],
          pairs_hbm.at[pl.ds(base_d + j * c_pairs, c_pairs)],
          out_sems[b],
      )

    def wait_out(b):
      pltpu.make_async_copy(
          bufb.at[b], pairs_hbm.at[pl.ds(base_d, c_pairs)], out_sems[b]
      ).wait()

    def repack(b):
      def body(p, carry):
        for l in range(embed_dim // LANES):
          bufb[b, 2 * p, pl.ds(l * LANES, LANES)] = (
              bufa[b, 4 * p, pl.ds(l * LANES, LANES)]
          )
          bufb[b, 2 * p, pl.ds(embed_dim + l * LANES, LANES)] = (
              bufa[b, 4 * p + 1, pl.ds(l * LANES, LANES)]
          )
          bufb[b, 2 * p + 1, pl.ds(l * LANES, LANES)] = (
              bufa[b, 4 * p + 2, pl.ds(l * LANES, LANES)]
          )
          bufb[b, 2 * p + 1, pl.ds(embed_dim + l * LANES, LANES)] = (
              bufa[b, 4 * p + 3, pl.ds(l * LANES, LANES)]
          )
        return carry

      lax.fori_loop(0, c_pairs // 2, body, 0)

    start_in(0, 0)
    start_in(1, 1)

    def step(s, carry):
      for b in range(2):
        j = 2 * s + b
        wait_in(b)

        @pl.when(j >= 2)
        def _():
          wait_out(b)

        repack(b)

        @pl.when(j + 2 < n_chunks)
        def _():
          start_in(j + 2, b)

        start_out(j, b)
      return carry

    lax.fori_loop(0, n_chunks // 2, step, 0)
    j = n_chunks - 1
    b = j % 2
    wait_in(b)
    wait_out(b)
    repack(b)
    start_out(j, b)
    wait_out(1 - b)
    wait_out(b)

    @pl.when(wid < extra)
    def _():
      r_d = base_d + n_chunks * c_pairs
      pltpu.sync_copy(
          table_hbm.at[pl.ds(2 * r_d, 16)], bufa.at[0, pl.ds(0, 16)]
      )
      for p in range(8):
        for l in range(embed_dim // LANES):
          bufb[0, p, pl.ds(l * LANES, LANES)] = (
              bufa[0, 2 * p, pl.ds(l * LANES, LANES)]
          )
          bufb[0, p, pl.ds(embed_dim + l * LANES, LANES)] = (
              bufa[0, 2 * p + 1, pl.ds(l * LANES, LANES)]
          )
      pltpu.sync_copy(bufb.at[0, pl.ds(0, 8)], pairs_hbm.at[pl.ds(r_d, 8)])

  return pack_kernel


@functools.cache
def _build_gather(embed_dim: int, batch_n: int):
  info, nw, mesh = _mesh_info()
  d2 = 2 * embed_dim
  b_per_w = batch_n // nw
  n_chunks = b_per_w // CHUNK

  @functools.partial(
      pl.kernel,
      mesh=mesh,
      out_type=jax.ShapeDtypeStruct((batch_n, embed_dim), jnp.float32),
      scratch_types=[
          pltpu.VMEM((n_chunks, CHUNK), jnp.int32),
          pltpu.VMEM((n_chunks, CHUNK), jnp.int32),
          pltpu.VMEM((2, CHUNK, d2), jnp.float32),
          pltpu.VMEM((b_per_w, embed_dim), jnp.float32),
          [pltpu.SemaphoreType.DMA] * 2,
      ],
  )
  def gather_kernel(pairs_hbm, idx_hbm, out_hbm, idx_v, pidx_v, rows_v,
                    stage_v, sems):
    wid = lax.axis_index("s") * info.num_cores + lax.axis_index("c")
    base = wid * b_per_w
    pltpu.sync_copy(idx_hbm.at[wid], idx_v)

    def quant(g, carry):
      j, gg = g // (CHUNK // LANES), g % (CHUNK // LANES)
      vec = idx_v[j, pl.ds(gg * LANES, LANES)]
      pidx_v[j, pl.ds(gg * LANES, LANES)] = lax.shift_right_logical(vec, 1)
      return carry

    lax.fori_loop(0, b_per_w // LANES, quant, 0)

    def start_gather(j, b):
      pltpu.async_copy(
          pairs_hbm.at[pidx_v.at[j]], rows_v.at[b], sems[b]
      )

    def wait_gather(b):
      pltpu.make_async_copy(
          pairs_hbm.at[pidx_v.at[0]], rows_v.at[b], sems[b]
      ).wait()

    start_gather(0, 0)
    for j in range(n_chunks):
      b = j % 2
      if j + 1 < n_chunks:
        start_gather(j + 1, 1 - b)
      wait_gather(b)

      def extract(g, carry):
        vec = idx_v[j, pl.ds(g * LANES, LANES)]
        for lane in range(LANES):
          k = g * LANES + lane
          half = lax.rem(vec[lane], 2) * embed_dim
          for l in range(embed_dim // LANES):
            stage_v[j * CHUNK + k, pl.ds(l * LANES, LANES)] = (
                rows_v[b, k, pl.ds(half + l * LANES, LANES)]
            )
        return carry

      lax.fori_loop(0, CHUNK // LANES, extract, 0)

    pltpu.sync_copy(stage_v, out_hbm.at[pl.ds(base, b_per_w)])

  return gather_kernel


def kernel(W, batch):
  total_nodes, embed_dim = W.shape
  (batch_n,) = batch.shape
  info = plsc.get_sparse_core_info()
  nw = info.num_cores * info.num_subcores
  pairs = _build_pack(embed_dim)(W)
  idx = batch.astype(jnp.int32).reshape(nw, batch_n // nw // CHUNK, CHUNK)
  return _build_gather(embed_dim, batch_n)(pairs, idx)


# speedup vs baseline: 1.5780x; 1.5780x over previous
import functools

import jax
import jax.numpy as jnp
from jax import lax
from jax.experimental import pallas as pl
from jax.experimental.pallas import tpu as pltpu
from jax.experimental.pallas import tpu_sc as plsc


@functools.cache
def _build(total_nodes: int, embed_dim: int, batch_n: int):
  info = plsc.get_sparse_core_info()
  nc, ns = info.num_cores, info.num_subcores
  nw = nc * ns
  b_per_w = batch_n // nw
  half = b_per_w // 2
  mesh = plsc.VectorSubcoreMesh(core_axis_name="c", subcore_axis_name="s")

  @functools.partial(
      pl.kernel,
      mesh=mesh,
      out_type=jax.ShapeDtypeStruct((batch_n, embed_dim), jnp.float32),
      scratch_types=[
          pltpu.VMEM((b_per_w,), jnp.int32),
          pltpu.VMEM((half, embed_dim), jnp.float32),
          pltpu.VMEM_SHARED((ns, half, embed_dim), jnp.float32),
          [pltpu.SemaphoreType.DMA] * 2,
      ],
  )
  def gather_kernel(table_hbm, idx_hbm, out_hbm, idx_v, rows_v, shared_v,
                    sems):
    sid = lax.axis_index("s")
    wid = sid * nc + lax.axis_index("c")
    base = wid * b_per_w
    pltpu.sync_copy(idx_hbm.at[pl.ds(base, b_per_w)], idx_v)

    def fire(g, carry):
      vec = idx_v[pl.ds(g * 16, 16)]
      for lane in range(16):
        k = g * 16 + lane

        @pl.when(k < half)
        def _():
          pltpu.async_copy(
              table_hbm.at[pl.ds(vec[lane], 1)],
              rows_v.at[pl.ds(k, 1)],
              sems[0],
          )

        @pl.when(k >= half)
        def _():
          pltpu.async_copy(
              table_hbm.at[pl.ds(vec[lane], 1)],
              shared_v.at[sid, pl.ds(k - half, 1)],
              sems[1],
          )
      return carry

    lax.fori_loop(0, b_per_w // 16, fire, 0)

    def drain(k, carry):
      pltpu.make_async_copy(
          table_hbm.at[pl.ds(0, 1)], rows_v.at[pl.ds(0, 1)], sems[0]
      ).wait()
      pltpu.make_async_copy(
          table_hbm.at[pl.ds(0, 1)], shared_v.at[0, pl.ds(0, 1)], sems[1]
      ).wait()
      return carry

    lax.fori_loop(0, half, drain, 0)
    pltpu.sync_copy(rows_v, out_hbm.at[pl.ds(base, half)])
    pltpu.sync_copy(shared_v.at[sid], out_hbm.at[pl.ds(base + half, half)])

  return gather_kernel


def kernel(W, batch):
  total_nodes, embed_dim = W.shape
  (batch_n,) = batch.shape
  gather_kernel = _build(total_nodes, embed_dim, batch_n)
  return gather_kernel(W, batch.astype(jnp.int32))


# speedup vs baseline: 1.6386x; 1.0384x over previous
import functools

import jax
import jax.numpy as jnp
from jax import lax
from jax.experimental import pallas as pl
from jax.experimental.pallas import tpu as pltpu
from jax.experimental.pallas import tpu_sc as plsc

NSEM = 4


@functools.cache
def _build(total_nodes: int, embed_dim: int, batch_n: int):
  info = plsc.get_sparse_core_info()
  nw = info.num_cores * info.num_subcores
  b_per_w = batch_n // nw
  mesh = plsc.VectorSubcoreMesh(core_axis_name="c", subcore_axis_name="s")

  @functools.partial(
      pl.kernel,
      mesh=mesh,
      out_type=jax.ShapeDtypeStruct((batch_n, embed_dim), jnp.float32),
      scratch_types=[
          pltpu.VMEM((b_per_w,), jnp.int32),
          pltpu.VMEM((b_per_w, embed_dim), jnp.float32),
          [pltpu.SemaphoreType.DMA] * NSEM,
      ],
  )
  def gather_kernel(table_hbm, idx_hbm, out_hbm, idx_v, rows_v, sems):
    wid = lax.axis_index("s") * info.num_cores + lax.axis_index("c")
    base = wid * b_per_w
    pltpu.sync_copy(idx_hbm.at[pl.ds(base, b_per_w)], idx_v)

    def fire(g, carry):
      vec = idx_v[pl.ds(g * 16, 16)]
      for lane in range(16):
        pltpu.async_copy(
            table_hbm.at[pl.ds(vec[lane], 1)],
            rows_v.at[pl.ds(g * 16 + lane, 1)],
            sems[lane % NSEM],
        )
      return carry

    lax.fori_loop(0, b_per_w // 16, fire, 0)

    def drain(k, carry):
      for s in range(NSEM):
        pltpu.make_async_copy(
            table_hbm.at[pl.ds(0, 1)], rows_v.at[pl.ds(0, 1)], sems[s]
        ).wait()
      return carry

    lax.fori_loop(0, b_per_w // NSEM, drain, 0)
    pltpu.sync_copy(rows_v, out_hbm.at[pl.ds(base, b_per_w)])

  return gather_kernel


def kernel(W, batch):
  total_nodes, embed_dim = W.shape
  (batch_n,) = batch.shape
  gather_kernel = _build(total_nodes, embed_dim, batch_n)
  return gather_kernel(W, batch.astype(jnp.int32))
